# baseline (speedup 1.0000x reference)
import functools

import jax
import jax.numpy as jnp
from jax import lax
from jax.experimental import pallas as pl
from jax.experimental.pallas import tpu as pltpu
from jax.experimental.pallas import tpu_sc as plsc

_CLAMP = 5.0


def _sc_info():
    try:
        info = plsc.get_sparse_core_info()
        return int(info.num_cores), int(info.num_subcores)
    except Exception:
        return 2, 16


def _qkv_body(x_ref, w_ref, qb_ref, q_ref, k_ref, v_ref):
    d = q_ref.shape[1]
    y = jnp.dot(x_ref[...], w_ref[...], preferred_element_type=jnp.float32)
    q_ref[...] = y[:, :d] + qb_ref[...]
    k_ref[...] = y[:, d:2 * d]
    v_ref[...] = y[:, 2 * d:]


def _edge_body(ea_ref, kq_ref, vs_ref, ewp_ref, ebp_ref, oew_ref,
               oeb_ref, a2p_ref, rb_ref,
               e1_ref, p128_ref, msgl_ref, msgr_ref, rtl_ref, rtr_ref, bnp_ref):
    ea = ea_ref[...]
    d = ea.shape[1]
    kq = kq_ref[...]
    ee = jnp.dot(ea.astype(jnp.bfloat16), ewp_ref[...],
                 preferred_element_type=jnp.float32) + ebp_ref[...]
    s = kq * ee[:, :d]
    s = jnp.sign(s) * jnp.sqrt(jnp.abs(s))
    et = jnp.maximum(s + ee[:, d:], 0.0)
    s128 = jnp.dot(et, a2p_ref[...], preferred_element_type=jnp.float32)
    p128 = jnp.exp(jnp.clip(s128, -_CLAMP, _CLAMP))
    p128_ref[...] = p128
    p256 = jnp.dot(p128, rb_ref[...], preferred_element_type=jnp.float32)
    msg = vs_ref[...] * p256
    rt = et * p256
    hd = d // 2
    msgl_ref[...] = msg[:, :hd]
    msgr_ref[...] = msg[:, hd:]
    rtl_ref[...] = rt[:, :hd]
    rtr_ref[...] = rt[:, hd:]
    e1 = ea + jnp.dot(et.astype(jnp.bfloat16), oew_ref[...],
                      preferred_element_type=jnp.float32) + oeb_ref[...]
    e1_ref[...] = e1
    bnp_ref[0, 0, :] = jnp.sum(e1, axis=0)
    bnp_ref[0, 1, :] = jnp.sum(e1 * e1, axis=0)


def _bnstats_body(cnt, bnp_ref, g_ref, b_ref, out_ref):
    t = jnp.sum(bnp_ref[...], axis=0)
    mu = t[0:1, :] * (1.0 / cnt)
    var = t[1:2, :] * (1.0 / cnt) - mu * mu
    istd = lax.rsqrt(var + 1e-5)
    scale = g_ref[...] * istd
    out_ref[0:1, :] = scale
    out_ref[1:2, :] = b_ref[...] - mu * scale


def _bnapply_body(x_ref, ss_ref, out_ref):
    out_ref[...] = x_ref[...] * ss_ref[0:1, :] + ss_ref[1:2, :]


def _f1_body(x_ref, a16a_ref, a16b_ref, accv_ref, accr_ref, vr2_ref, ohw_ref,
             ohb_ref, c0_ref, c1_ref, rb_ref, d8_ref, h1_ref, bnp_ref):
    a16 = a16a_ref[...] + a16b_ref[...]
    r256 = jnp.dot(1.0 / (a16 + 1e-16), rb_ref[...],
                   preferred_element_type=jnp.float32)
    deg256 = jnp.dot(a16, d8_ref[...], preferred_element_type=jnp.float32)
    ld = jnp.log(deg256 + 1.0)
    wvt = (accv_ref[...]
           + jnp.dot(accr_ref[...], vr2_ref[...], preferred_element_type=jnp.float32)) * r256
    hh = wvt * (c0_ref[...] + ld * c1_ref[...])
    h1 = x_ref[...] + jnp.dot(hh, ohw_ref[...], preferred_element_type=jnp.float32) + ohb_ref[...]
    h1_ref[...] = h1
    bnp_ref[0, 0, :] = jnp.sum(h1, axis=0)
    bnp_ref[0, 1, :] = jnp.sum(h1 * h1, axis=0)


def _f2_body(cnt, h1_ref, bnp1_ref, g_ref, b_ref, w1_ref, bb1_ref, w2_ref,
             bb2_ref, f_ref, bnp2_ref):
    t = jnp.sum(bnp1_ref[...], axis=0)
    mu = t[0:1, :] * (1.0 / cnt)
    var = t[1:2, :] * (1.0 / cnt) - mu * mu
    istd = lax.rsqrt(var + 1e-5)
    scale = g_ref[...] * istd
    shift = b_ref[...] - mu * scale
    hn = h1_ref[...] * scale + shift
    ff = jnp.maximum(jnp.dot(hn, w1_ref[...], preferred_element_type=jnp.float32)
                     + bb1_ref[...], 0.0)
    f = hn + jnp.dot(ff, w2_ref[...], preferred_element_type=jnp.float32) + bb2_ref[...]
    f_ref[...] = f
    bnp2_ref[0, 0, :] = jnp.sum(f, axis=0)
    bnp2_ref[0, 1, :] = jnp.sum(f * f, axis=0)


def _f3_body(cnt, f_ref, bnp2_ref, g_ref, b_ref, out_ref):
    t = jnp.sum(bnp2_ref[...], axis=0)
    mu = t[0:1, :] * (1.0 / cnt)
    var = t[1:2, :] * (1.0 / cnt) - mu * mu
    istd = lax.rsqrt(var + 1e-5)
    scale = g_ref[...] * istd
    shift = b_ref[...] - mu * scale
    out_ref[...] = f_ref[...] * scale + shift


def _sc_gather(qh, kh, vh, src2d, dst2d):
    n, d = qh.shape
    ch = src2d.shape[1]
    rows = src2d.shape[0] - 8
    e = rows * ch
    nc, ns = _sc_info()
    nw = nc * ns
    q, r = divmod(rows, nw)
    qload = ((q + (1 if r else 0) + 8 + 7) // 8) * 8
    mesh = plsc.VectorSubcoreMesh(core_axis_name="c", subcore_axis_name="s")

    @functools.partial(
        pl.kernel,
        out_type=(jax.ShapeDtypeStruct((e, d), jnp.float32),
                  jax.ShapeDtypeStruct((e, d), jnp.float32)),
        mesh=mesh,
        scratch_types=[
            pltpu.VMEM((qload, ch), jnp.int32),
            pltpu.VMEM((qload, ch), jnp.int32),
            pltpu.VMEM((ch, d), jnp.float32),
            pltpu.VMEM((ch, d), jnp.float32),
            pltpu.VMEM((ch, d), jnp.float32),
            pltpu.VMEM((ch, d), jnp.float32),
            pltpu.VMEM((ch, d), jnp.float32),
            pltpu.VMEM((ch, d), jnp.float32),
            pltpu.SemaphoreType.DMA,
            pltpu.SemaphoreType.DMA,
            pltpu.SemaphoreType.DMA,
            pltpu.SemaphoreType.DMA,
        ],
    )
    def gat(qh_h, kh_h, vh_h, s2_h, d2_h, kq_o, vs_o,
            isall, idall, kb0, qb0, vb0, kb1, qb1, vb1, gs0, gs1, ws0, ws1):
        wid = lax.axis_index("c") * ns + lax.axis_index("s")
        c0 = wid * q + jnp.maximum(wid - (nw - r), 0) if r else wid * q
        nch = (q + jnp.where(wid >= nw - r, 1, 0)) if r else q
        base8 = (c0 // 8) * 8
        off = c0 - base8
        pltpu.sync_copy(s2_h.at[pl.ds(base8, qload)], isall)
        pltpu.sync_copy(d2_h.at[pl.ds(base8, qload)], idall)

        def issue_g(i, kb, qb, vb, gs):
            pltpu.async_copy(kh_h.at[isall.at[off + i]], kb, gs)
            pltpu.async_copy(qh_h.at[idall.at[off + i]], qb, gs)
            pltpu.async_copy(vh_h.at[isall.at[off + i]], vb, gs)

        def wait_g(i, kb, qb, vb, gs):
            pltpu.make_async_copy(kh_h.at[isall.at[off + i]], kb, gs).wait()
            pltpu.make_async_copy(qh_h.at[idall.at[off + i]], qb, gs).wait()
            pltpu.make_async_copy(vh_h.at[isall.at[off + i]], vb, gs).wait()

        def issue_w(i, kb, qb, vb, ws):
            base = (c0 + i) * ch
            pltpu.async_copy(kb, kq_o.at[pl.ds(base, ch)], ws)
            pltpu.async_copy(vb, vs_o.at[pl.ds(base, ch)], ws)

        def wait_w(kb, qb, vb, ws):
            pltpu.make_async_copy(kb, kq_o.at[pl.ds(0, ch)], ws).wait()
            pltpu.make_async_copy(vb, vs_o.at[pl.ds(0, ch)], ws).wait()

        def addq(kb, qb):
            def rowfn(j, carry):
                for t in range(d // 16):
                    sl = pl.ds(t * 16, 16)
                    kb[j, sl] = kb[j, sl] + qb[j, sl]
                return carry

            lax.fori_loop(0, ch, rowfn, 0)

        issue_g(0, kb0, qb0, vb0, gs0)

        def step(i, kbC, qbC, vbC, gsC, wsC, kbO, qbO, vbO, gsO, wsO):
            @pl.when(i + 1 < nch)
            def _():
                @pl.when(i >= 1)
                def _():
                    wait_w(kbO, qbO, vbO, wsO)
                issue_g(i + 1, kbO, qbO, vbO, gsO)
            wait_g(i, kbC, qbC, vbC, gsC)
            addq(kbC, qbC)
            issue_w(i, kbC, qbC, vbC, wsC)

        def body(i, carry):
            @pl.when(i % 2 == 0)
            def _():
                step(i, kb0, qb0, vb0, gs0, ws0, kb1, qb1, vb1, gs1, ws1)

            @pl.when(i % 2 == 1)
            def _():
                step(i, kb1, qb1, vb1, gs1, ws1, kb0, qb0, vb0, gs0, ws0)

            return carry

        lax.fori_loop(0, nch, body, 0)
        wait_w(kb0, qb0, vb0, ws0)
        wait_w(kb1, qb1, vb1, ws1)

    return gat(qh, kh, vh, src2d, dst2d)


def _scatter_loop(data_h, acc, idxall, dbuf0, dbuf1, sem0, sem1,
                  c0, off, nch):
    pltpu.async_copy(data_h.at[pl.ds(c0 * 128, 128)], dbuf0, sem0)

    def step(i, cur, scur, nxt, snxt):
        @pl.when(i + 1 < nch)
        def _():
            pltpu.async_copy(data_h.at[pl.ds((c0 + i + 1) * 128, 128)],
                             nxt, snxt)
        pltpu.make_async_copy(data_h.at[pl.ds((c0 + i) * 128, 128)],
                              cur, scur).wait()
        pltpu.sync_copy(cur, acc.at[idxall.at[off + i]], add=True)

    def body(i, carry):
        @pl.when(i % 2 == 0)
        def _():
            step(i, dbuf0, sem0, dbuf1, sem1)

        @pl.when(i % 2 == 1)
        def _():
            step(i, dbuf1, sem1, dbuf0, sem0)

        return carry

    lax.fori_loop(0, nch, body, 0)


def _acc_writeout(acc, out_h, cid, sid, n, zrows):
    full_t = n // zrows
    rem = n - full_t * zrows

    @pl.when(sid < full_t)
    def _():
        pltpu.sync_copy(acc.at[pl.ds(sid * zrows, zrows)],
                        out_h.at[cid, pl.ds(sid * zrows, zrows)])

    if rem:
        @pl.when(sid == full_t)
        def _():
            pltpu.sync_copy(acc.at[pl.ds(full_t * zrows, rem)],
                            out_h.at[cid, pl.ds(full_t * zrows, rem)])


def _sc_scatter_part(p128, dst2d, n):
    e, w = p128.shape
    nc, ns = _sc_info()
    nw = nc * ns
    nch_total = e // 128
    q, r = divmod(nch_total, nw)
    qload = ((q + (1 if r else 0) + 8 + 7) // 8) * 8
    npad = ((n + nw * 8 - 1) // (nw * 8)) * (nw * 8)
    zrows = npad // ns
    zeros = jnp.zeros((zrows, w), jnp.float32)
    mesh = plsc.VectorSubcoreMesh(core_axis_name="c", subcore_axis_name="s")

    @functools.partial(
        pl.kernel,
        out_type=jax.ShapeDtypeStruct((nc, n, w), jnp.float32),
        mesh=mesh,
        scratch_types=[
            pltpu.VMEM((qload, 128), jnp.int32),
            pltpu.VMEM((128, w), jnp.float32),
            pltpu.VMEM((128, w), jnp.float32),
            pltpu.VMEM_SHARED((npad, w), jnp.float32),
            pltpu.SemaphoreType.DMA,
            pltpu.SemaphoreType.DMA,
        ],
    )
    def sca(p_h, dst2d_h, z_h, out_h, idxall, dbuf0, dbuf1, acc, sem0, sem1):
        cid = lax.axis_index("c")
        sid = lax.axis_index("s")
        wid = cid * ns + sid
        pltpu.sync_copy(z_h, acc.at[pl.ds(sid * zrows, zrows)])
        c0 = wid * q + jnp.maximum(wid - (nw - r), 0) if r else wid * q
        nch = q + jnp.where(wid >= nw - r, 1, 0) if r else q
        base8 = (c0 // 8) * 8
        off = c0 - base8
        pltpu.sync_copy(dst2d_h.at[pl.ds(base8, qload)], idxall)
        plsc.subcore_barrier()
        _scatter_loop(p_h, acc, idxall, dbuf0, dbuf1, sem0, sem1,
                      c0, off, nch)
        plsc.subcore_barrier()
        _acc_writeout(acc, out_h, cid, sid, n, zrows)

    return sca(p128, dst2d, zeros)


def _sc_scatter128(ml, mr, dst2d, n):
    e, hd = ml.shape
    nc, ns = _sc_info()
    nw = nc * ns
    nch_total = e // 128
    q, r = divmod(nch_total, ns)
    qload = ((q + (1 if r else 0) + 8 + 7) // 8) * 8
    npad = ((n + nw * 8 - 1) // (nw * 8)) * (nw * 8)
    zrows = npad // ns
    zeros = jnp.zeros((zrows, hd), jnp.float32)
    mesh = plsc.VectorSubcoreMesh(core_axis_name="c", subcore_axis_name="s")

    @functools.partial(
        pl.kernel,
        out_type=jax.ShapeDtypeStruct((nc, n, hd), jnp.float32),
        mesh=mesh,
        scratch_types=[
            pltpu.VMEM((qload, 128), jnp.int32),
            pltpu.VMEM((128, hd), jnp.float32),
            pltpu.VMEM((128, hd), jnp.float32),
            pltpu.VMEM_SHARED((npad, hd), jnp.float32),
            pltpu.SemaphoreType.DMA,
            pltpu.SemaphoreType.DMA,
        ],
    )
    def sca(ml_h, mr_h, dst2d_h, z_h, out_h, idxall, dbuf0, dbuf1, acc,
            sem0, sem1):
        cid = lax.axis_index("c")
        sid = lax.axis_index("s")
        pltpu.sync_copy(z_h, acc.at[pl.ds(sid * zrows, zrows)])
        c0 = sid * q + jnp.maximum(sid - (ns - r), 0) if r else sid * q
        nch = q + jnp.where(sid >= ns - r, 1, 0) if r else q
        base8 = (c0 // 8) * 8
        off = c0 - base8
        pltpu.sync_copy(dst2d_h.at[pl.ds(base8, qload)], idxall)
        plsc.subcore_barrier()

        @pl.when(cid == 0)
        def _():
            _scatter_loop(ml_h, acc, idxall, dbuf0, dbuf1,
                          sem0, sem1, c0, off, nch)

        @pl.when(cid == 1)
        def _():
            _scatter_loop(mr_h, acc, idxall, dbuf0, dbuf1,
                          sem0, sem1, c0, off, nch)

        plsc.subcore_barrier()
        _acc_writeout(acc, out_h, cid, sid, n, zrows)

    return sca(ml, mr, dst2d, zeros)


def kernel(x, edge_attr, edge_index, Qw, Qb, Kw, Ew, Eb, Vw, Aw, VeRow,
           deg_coef, Ohw, Ohb, Oew, Oeb, bn1h_g, bn1h_b, bn1e_g, bn1e_b,
           W1, b1, W2, b2, bn2h_g, bn2h_b):
    n, d = x.shape
    e = edge_attr.shape[0]
    dh, h, _ = Aw.shape
    f32 = jnp.float32

    src = edge_index[0]
    dst = edge_index[1]

    w3 = jnp.concatenate([Qw, Kw, Vw], axis=1)
    qb2 = Qb.reshape(1, d)
    ewp = Ew.reshape(d, h, 2, dh).transpose(0, 2, 1, 3).reshape(d, 2 * d)
    ebp = Eb.reshape(h, 2, dh).transpose(1, 0, 2).reshape(1, 2 * d)
    eye_h = jnp.eye(h, dtype=f32)
    a2 = (Aw[:, :, 0][:, :, None] * eye_h[None, :, :]).transpose(1, 0, 2).reshape(d, h)
    a2p = jnp.concatenate([a2, jnp.zeros((d, 128 - h), f32)], axis=1)
    rb8 = jnp.kron(eye_h, jnp.ones((1, dh), f32))
    rb = jnp.concatenate([rb8, jnp.zeros((128 - h, d), f32)], axis=0)
    d8 = jnp.zeros((128, d), f32).at[h].set(1.0)
    vr2 = (VeRow[:, :, None, :] * eye_h[None, :, :, None]).transpose(1, 0, 2, 3).reshape(d, d)
    c0 = deg_coef[0, :, 0].reshape(1, d)
    c1 = deg_coef[0, :, 1].reshape(1, d)
    oebr = Oeb.reshape(1, d)
    ohbr = Ohb.reshape(1, d)

    bn = 1000
    gn = n // bn
    qh, kh, vh = pl.pallas_call(
        _qkv_body,
        grid=(gn,),
        in_specs=[pl.BlockSpec((bn, d), lambda i: (i, 0)),
                  pl.BlockSpec((d, 3 * d), lambda i: (0, 0)),
                  pl.BlockSpec((1, d), lambda i: (0, 0))],
        out_specs=[pl.BlockSpec((bn, d), lambda i: (i, 0))] * 3,
        out_shape=[jax.ShapeDtypeStruct((n, d), f32)] * 3,
    )(x, w3, qb2)

    pad8 = jnp.zeros((8, 64), jnp.int32)
    src2d = jnp.concatenate([src.reshape(e // 64, 64), pad8], axis=0)
    dst2dg = jnp.concatenate([dst.reshape(e // 64, 64), pad8], axis=0)
    kq, vs = _sc_gather(qh, kh, vh, src2d, dst2dg)

    be = 1280
    ge = e // be
    hd = d // 2
    e1, p128, msgl, msgr, rtl, rtr, bnpe = pl.pallas_call(
        _edge_body,
        grid=(ge,),
        in_specs=[pl.BlockSpec((be, d), lambda i: (i, 0)),
                  pl.BlockSpec((be, d), lambda i: (i, 0)),
                  pl.BlockSpec((be, d), lambda i: (i, 0)),
                  pl.BlockSpec((d, 2 * d), lambda i: (0, 0)),
                  pl.BlockSpec((1, 2 * d), lambda i: (0, 0)),
                  pl.BlockSpec((d, d), lambda i: (0, 0)),
                  pl.BlockSpec((1, d), lambda i: (0, 0)),
                  pl.BlockSpec((d, 128), lambda i: (0, 0)),
                  pl.BlockSpec((128, d), lambda i: (0, 0))],
        out_specs=[pl.BlockSpec((be, d), lambda i: (i, 0)),
                   pl.BlockSpec((be, 128), lambda i: (i, 0)),
                   pl.BlockSpec((be, hd), lambda i: (i, 0)),
                   pl.BlockSpec((be, hd), lambda i: (i, 0)),
                   pl.BlockSpec((be, hd), lambda i: (i, 0)),
                   pl.BlockSpec((be, hd), lambda i: (i, 0)),
                   pl.BlockSpec((1, 2, d), lambda i: (i, 0, 0))],
        out_shape=[jax.ShapeDtypeStruct((e, d), f32),
                   jax.ShapeDtypeStruct((e, 128), f32),
                   jax.ShapeDtypeStruct((e, hd), f32),
                   jax.ShapeDtypeStruct((e, hd), f32),
                   jax.ShapeDtypeStruct((e, hd), f32),
                   jax.ShapeDtypeStruct((e, hd), f32),
                   jax.ShapeDtypeStruct((ge, 2, d), f32)],
    )(edge_attr, kq, vs, ewp.astype(jnp.bfloat16), ebp,
      Oew.astype(jnp.bfloat16), oebr, a2p, rb)

    dst2d = jnp.concatenate(
        [dst.reshape(e // 128, 128),
         jnp.zeros((8, 128), jnp.int32)], axis=0)
    acc16 = _sc_scatter_part(p128, dst2d, n)
    accv2 = _sc_scatter128(msgl, msgr, dst2d, n)
    accr2 = _sc_scatter128(rtl, rtr, dst2d, n)
    accv = jnp.concatenate([accv2[0], accv2[1]], axis=1)
    accr = jnp.concatenate([accr2[0], accr2[1]], axis=1)

    ss_e = pl.pallas_call(
        functools.partial(_bnstats_body, float(e)),
        grid=(1,),
        in_specs=[pl.BlockSpec((ge, 2, d), lambda i: (0, 0, 0)),
                  pl.BlockSpec((1, d), lambda i: (0, 0)),
                  pl.BlockSpec((1, d), lambda i: (0, 0))],
        out_specs=pl.BlockSpec((2, d), lambda i: (0, 0)),
        out_shape=jax.ShapeDtypeStruct((2, d), f32),
    )(bnpe, bn1e_g.reshape(1, d), bn1e_b.reshape(1, d))
    e_out = pl.pallas_call(
        _bnapply_body,
        grid=(ge,),
        in_specs=[pl.BlockSpec((be, d), lambda i: (i, 0)),
                  pl.BlockSpec((2, d), lambda i: (0, 0))],
        out_specs=pl.BlockSpec((be, d), lambda i: (i, 0)),
        out_shape=jax.ShapeDtypeStruct((e, d), f32),
    )(e1, ss_e)

    h1, bnp1 = pl.pallas_call(
        _f1_body,
        grid=(gn,),
        in_specs=[pl.BlockSpec((bn, d), lambda i: (i, 0)),
                  pl.BlockSpec((bn, 128), lambda i: (i, 0)),
                  pl.BlockSpec((bn, 128), lambda i: (i, 0)),
                  pl.BlockSpec((bn, d), lambda i: (i, 0)),
                  pl.BlockSpec((bn, d), lambda i: (i, 0)),
                  pl.BlockSpec((d, d), lambda i: (0, 0)),
                  pl.BlockSpec((d, d), lambda i: (0, 0)),
                  pl.BlockSpec((1, d), lambda i: (0, 0)),
                  pl.BlockSpec((1, d), lambda i: (0, 0)),
                  pl.BlockSpec((1, d), lambda i: (0, 0)),
                  pl.BlockSpec((128, d), lambda i: (0, 0)),
                  pl.BlockSpec((128, d), lambda i: (0, 0))],
        out_specs=[pl.BlockSpec((bn, d), lambda i: (i, 0)),
                   pl.BlockSpec((1, 2, d), lambda i: (i, 0, 0))],
        out_shape=[jax.ShapeDtypeStruct((n, d), f32),
                   jax.ShapeDtypeStruct((gn, 2, d), f32)],
    )(x, acc16[0], acc16[1], accv, accr, vr2, Ohw, ohbr, c0, c1, rb, d8)

    f_mid, bnp2 = pl.pallas_call(
        functools.partial(_f2_body, float(n)),
        grid=(gn,),
        in_specs=[pl.BlockSpec((bn, d), lambda i: (i, 0)),
                  pl.BlockSpec((gn, 2, d), lambda i: (0, 0, 0)),
                  pl.BlockSpec((1, d), lambda i: (0, 0)),
                  pl.BlockSpec((1, d), lambda i: (0, 0)),
                  pl.BlockSpec((d, 2 * d), lambda i: (0, 0)),
                  pl.BlockSpec((1, 2 * d), lambda i: (0, 0)),
                  pl.BlockSpec((2 * d, d), lambda i: (0, 0)),
                  pl.BlockSpec((1, d), lambda i: (0, 0))],
        out_specs=[pl.BlockSpec((bn, d), lambda i: (i, 0)),
                   pl.BlockSpec((1, 2, d), lambda i: (i, 0, 0))],
        out_shape=[jax.ShapeDtypeStruct((n, d), f32),
                   jax.ShapeDtypeStruct((gn, 2, d), f32)],
    )(h1, bnp1, bn1h_g.reshape(1, d), bn1h_b.reshape(1, d), W1,
      b1.reshape(1, 2 * d), W2, b2.reshape(1, d))

    h_out = pl.pallas_call(
        functools.partial(_f3_body, float(n)),
        grid=(gn,),
        in_specs=[pl.BlockSpec((bn, d), lambda i: (i, 0)),
                  pl.BlockSpec((gn, 2, d), lambda i: (0, 0, 0)),
                  pl.BlockSpec((1, d), lambda i: (0, 0)),
                  pl.BlockSpec((1, d), lambda i: (0, 0))],
        out_specs=pl.BlockSpec((bn, d), lambda i: (i, 0)),
        out_shape=jax.ShapeDtypeStruct((n, d), f32),
    )(f_mid, bnp2, bn2h_g.reshape(1, d), bn2h_b.reshape(1, d))

    return h_out, e_out


# speedup vs baseline: 1.0024x; 1.0024x over previous
import functools

import jax
import jax.numpy as jnp
from jax import lax
from jax.experimental import pallas as pl
from jax.experimental.pallas import tpu as pltpu
from jax.experimental.pallas import tpu_sc as plsc

_CLAMP = 5.0


def _sc_info():
    try:
        info = plsc.get_sparse_core_info()
        return int(info.num_cores), int(info.num_subcores)
    except Exception:
        return 2, 16


def _qkv_body(x_ref, w_ref, qb_ref, q_ref, k_ref, v_ref):
    d = q_ref.shape[1]
    y = jnp.dot(x_ref[...], w_ref[...], preferred_element_type=jnp.float32)
    q_ref[...] = y[:, :d] + qb_ref[...]
    k_ref[...] = y[:, d:2 * d]
    v_ref[...] = y[:, 2 * d:]


def _edge_body(ea_ref, kq_ref, vs_ref, ewp_ref, ebp_ref, oew_ref,
               oeb_ref, a2p_ref, rb_ref,
               e1_ref, p128_ref, msgl_ref, msgr_ref, rtl_ref, rtr_ref, bnp_ref):
    ea = ea_ref[...]
    d = ea.shape[1]
    kq = kq_ref[...]
    ee = jnp.dot(ea, ewp_ref[...], preferred_element_type=jnp.float32) + ebp_ref[...]
    s = kq * ee[:, :d]
    s = jnp.sign(s) * jnp.sqrt(jnp.abs(s))
    et = jnp.maximum(s + ee[:, d:], 0.0)
    s128 = jnp.dot(et, a2p_ref[...], preferred_element_type=jnp.float32)
    p128 = jnp.exp(jnp.clip(s128, -_CLAMP, _CLAMP))
    p128_ref[...] = p128
    p256 = jnp.dot(p128, rb_ref[...], preferred_element_type=jnp.float32)
    msg = vs_ref[...] * p256
    rt = et * p256
    hd = d // 2
    msgl_ref[...] = msg[:, :hd]
    msgr_ref[...] = msg[:, hd:]
    rtl_ref[...] = rt[:, :hd]
    rtr_ref[...] = rt[:, hd:]
    e1 = ea + jnp.dot(et, oew_ref[...], preferred_element_type=jnp.float32) + oeb_ref[...]
    e1_ref[...] = e1
    bnp_ref[0, 0, :] = jnp.sum(e1, axis=0)
    bnp_ref[0, 1, :] = jnp.sum(e1 * e1, axis=0)


def _bnstats_body(cnt, bnp_ref, g_ref, b_ref, out_ref):
    t = jnp.sum(bnp_ref[...], axis=0)
    mu = t[0:1, :] * (1.0 / cnt)
    var = t[1:2, :] * (1.0 / cnt) - mu * mu
    istd = lax.rsqrt(var + 1e-5)
    scale = g_ref[...] * istd
    out_ref[0:1, :] = scale
    out_ref[1:2, :] = b_ref[...] - mu * scale


def _bnapply_body(x_ref, ss_ref, out_ref):
    out_ref[...] = x_ref[...] * ss_ref[0:1, :] + ss_ref[1:2, :]


def _f1_body(x_ref, a16a_ref, a16b_ref, accv_ref, accr_ref, vr2_ref, ohw_ref,
             ohb_ref, c0_ref, c1_ref, rb_ref, d8_ref, h1_ref, bnp_ref):
    a16 = a16a_ref[...] + a16b_ref[...]
    r256 = jnp.dot(1.0 / (a16 + 1e-16), rb_ref[...],
                   preferred_element_type=jnp.float32)
    deg256 = jnp.dot(a16, d8_ref[...], preferred_element_type=jnp.float32)
    ld = jnp.log(deg256 + 1.0)
    wvt = (accv_ref[...]
           + jnp.dot(accr_ref[...], vr2_ref[...], preferred_element_type=jnp.float32)) * r256
    hh = wvt * (c0_ref[...] + ld * c1_ref[...])
    h1 = x_ref[...] + jnp.dot(hh, ohw_ref[...], preferred_element_type=jnp.float32) + ohb_ref[...]
    h1_ref[...] = h1
    bnp_ref[0, 0, :] = jnp.sum(h1, axis=0)
    bnp_ref[0, 1, :] = jnp.sum(h1 * h1, axis=0)


def _f2_body(cnt, h1_ref, bnp1_ref, g_ref, b_ref, w1_ref, bb1_ref, w2_ref,
             bb2_ref, f_ref, bnp2_ref):
    t = jnp.sum(bnp1_ref[...], axis=0)
    mu = t[0:1, :] * (1.0 / cnt)
    var = t[1:2, :] * (1.0 / cnt) - mu * mu
    istd = lax.rsqrt(var + 1e-5)
    scale = g_ref[...] * istd
    shift = b_ref[...] - mu * scale
    hn = h1_ref[...] * scale + shift
    ff = jnp.maximum(jnp.dot(hn, w1_ref[...], preferred_element_type=jnp.float32)
                     + bb1_ref[...], 0.0)
    f = hn + jnp.dot(ff, w2_ref[...], preferred_element_type=jnp.float32) + bb2_ref[...]
    f_ref[...] = f
    bnp2_ref[0, 0, :] = jnp.sum(f, axis=0)
    bnp2_ref[0, 1, :] = jnp.sum(f * f, axis=0)


def _f3_body(cnt, f_ref, bnp2_ref, g_ref, b_ref, out_ref):
    t = jnp.sum(bnp2_ref[...], axis=0)
    mu = t[0:1, :] * (1.0 / cnt)
    var = t[1:2, :] * (1.0 / cnt) - mu * mu
    istd = lax.rsqrt(var + 1e-5)
    scale = g_ref[...] * istd
    shift = b_ref[...] - mu * scale
    out_ref[...] = f_ref[...] * scale + shift


def _sc_gather(qh, kh, vh, src2d, dst2d):
    n, d = qh.shape
    ch = src2d.shape[1]
    rows = src2d.shape[0] - 8
    e = rows * ch
    nc, ns = _sc_info()
    nw = nc * ns
    q, r = divmod(rows, nw)
    qload = ((q + (1 if r else 0) + 8 + 7) // 8) * 8
    mesh = plsc.VectorSubcoreMesh(core_axis_name="c", subcore_axis_name="s")

    @functools.partial(
        pl.kernel,
        out_type=(jax.ShapeDtypeStruct((e, d), jnp.float32),
                  jax.ShapeDtypeStruct((e, d), jnp.float32)),
        mesh=mesh,
        scratch_types=[
            pltpu.VMEM((qload, ch), jnp.int32),
            pltpu.VMEM((qload, ch), jnp.int32),
            pltpu.VMEM((ch, d), jnp.float32),
            pltpu.VMEM((ch, d), jnp.float32),
            pltpu.VMEM((ch, d), jnp.float32),
            pltpu.VMEM((ch, d), jnp.float32),
            pltpu.VMEM((ch, d), jnp.float32),
            pltpu.VMEM((ch, d), jnp.float32),
            pltpu.SemaphoreType.DMA,
            pltpu.SemaphoreType.DMA,
            pltpu.SemaphoreType.DMA,
            pltpu.SemaphoreType.DMA,
        ],
    )
    def gat(qh_h, kh_h, vh_h, s2_h, d2_h, kq_o, vs_o,
            isall, idall, kb0, qb0, vb0, kb1, qb1, vb1, gs0, gs1, ws0, ws1):
        wid = lax.axis_index("c") * ns + lax.axis_index("s")
        c0 = wid * q + jnp.maximum(wid - (nw - r), 0) if r else wid * q
        nch = (q + jnp.where(wid >= nw - r, 1, 0)) if r else q
        base8 = (c0 // 8) * 8
        off = c0 - base8
        pltpu.sync_copy(s2_h.at[pl.ds(base8, qload)], isall)
        pltpu.sync_copy(d2_h.at[pl.ds(base8, qload)], idall)

        def issue_g(i, kb, qb, vb, gs):
            pltpu.async_copy(kh_h.at[isall.at[off + i]], kb, gs)
            pltpu.async_copy(qh_h.at[idall.at[off + i]], qb, gs)
            pltpu.async_copy(vh_h.at[isall.at[off + i]], vb, gs)

        def wait_g(i, kb, qb, vb, gs):
            pltpu.make_async_copy(kh_h.at[isall.at[off + i]], kb, gs).wait()
            pltpu.make_async_copy(qh_h.at[idall.at[off + i]], qb, gs).wait()
            pltpu.make_async_copy(vh_h.at[isall.at[off + i]], vb, gs).wait()

        def issue_w(i, kb, qb, vb, ws):
            base = (c0 + i) * ch
            pltpu.async_copy(kb, kq_o.at[pl.ds(base, ch)], ws)
            pltpu.async_copy(vb, vs_o.at[pl.ds(base, ch)], ws)

        def wait_w(kb, qb, vb, ws):
            pltpu.make_async_copy(kb, kq_o.at[pl.ds(0, ch)], ws).wait()
            pltpu.make_async_copy(vb, vs_o.at[pl.ds(0, ch)], ws).wait()

        def addq(kb, qb):
            def rowfn(j, carry):
                for t in range(d // 16):
                    sl = pl.ds(t * 16, 16)
                    kb[j, sl] = kb[j, sl] + qb[j, sl]
                return carry

            lax.fori_loop(0, ch, rowfn, 0)

        issue_g(0, kb0, qb0, vb0, gs0)

        def step(i, kbC, qbC, vbC, gsC, wsC, kbO, qbO, vbO, gsO, wsO):
            @pl.when(i + 1 < nch)
            def _():
                @pl.when(i >= 1)
                def _():
                    wait_w(kbO, qbO, vbO, wsO)
                issue_g(i + 1, kbO, qbO, vbO, gsO)
            wait_g(i, kbC, qbC, vbC, gsC)
            addq(kbC, qbC)
            issue_w(i, kbC, qbC, vbC, wsC)

        def body(i, carry):
            @pl.when(i % 2 == 0)
            def _():
                step(i, kb0, qb0, vb0, gs0, ws0, kb1, qb1, vb1, gs1, ws1)

            @pl.when(i % 2 == 1)
            def _():
                step(i, kb1, qb1, vb1, gs1, ws1, kb0, qb0, vb0, gs0, ws0)

            return carry

        lax.fori_loop(0, nch, body, 0)
        wait_w(kb0, qb0, vb0, ws0)
        wait_w(kb1, qb1, vb1, ws1)

    return gat(qh, kh, vh, src2d, dst2d)


def _scatter_loop(data_h, acc, idxall, dbuf0, dbuf1, sem0, sem1,
                  c0, off, nch):
    pltpu.async_copy(data_h.at[pl.ds(c0 * 128, 128)], dbuf0, sem0)

    def step(i, cur, scur, nxt, snxt):
        @pl.when(i + 1 < nch)
        def _():
            pltpu.async_copy(data_h.at[pl.ds((c0 + i + 1) * 128, 128)],
                             nxt, snxt)
        pltpu.make_async_copy(data_h.at[pl.ds((c0 + i) * 128, 128)],
                              cur, scur).wait()
        pltpu.sync_copy(cur, acc.at[idxall.at[off + i]], add=True)

    def body(i, carry):
        @pl.when(i % 2 == 0)
        def _():
            step(i, dbuf0, sem0, dbuf1, sem1)

        @pl.when(i % 2 == 1)
        def _():
            step(i, dbuf1, sem1, dbuf0, sem0)

        return carry

    lax.fori_loop(0, nch, body, 0)


def _acc_writeout(acc, out_h, cid, sid, n, zrows):
    full_t = n // zrows
    rem = n - full_t * zrows

    @pl.when(sid < full_t)
    def _():
        pltpu.sync_copy(acc.at[pl.ds(sid * zrows, zrows)],
                        out_h.at[cid, pl.ds(sid * zrows, zrows)])

    if rem:
        @pl.when(sid == full_t)
        def _():
            pltpu.sync_copy(acc.at[pl.ds(full_t * zrows, rem)],
                            out_h.at[cid, pl.ds(full_t * zrows, rem)])


def _sc_scatter_part(p128, dst2d, n):
    e, w = p128.shape
    nc, ns = _sc_info()
    nw = nc * ns
    nch_total = e // 128
    q, r = divmod(nch_total, nw)
    qload = ((q + (1 if r else 0) + 8 + 7) // 8) * 8
    npad = ((n + nw * 8 - 1) // (nw * 8)) * (nw * 8)
    zrows = npad // ns
    zeros = jnp.zeros((zrows, w), jnp.float32)
    mesh = plsc.VectorSubcoreMesh(core_axis_name="c", subcore_axis_name="s")

    @functools.partial(
        pl.kernel,
        out_type=jax.ShapeDtypeStruct((nc, n, w), jnp.float32),
        mesh=mesh,
        scratch_types=[
            pltpu.VMEM((qload, 128), jnp.int32),
            pltpu.VMEM((128, w), jnp.float32),
            pltpu.VMEM((128, w), jnp.float32),
            pltpu.VMEM_SHARED((npad, w), jnp.float32),
            pltpu.SemaphoreType.DMA,
            pltpu.SemaphoreType.DMA,
        ],
    )
    def sca(p_h, dst2d_h, z_h, out_h, idxall, dbuf0, dbuf1, acc, sem0, sem1):
        cid = lax.axis_index("c")
        sid = lax.axis_index("s")
        wid = cid * ns + sid
        pltpu.sync_copy(z_h, acc.at[pl.ds(sid * zrows, zrows)])
        c0 = wid * q + jnp.maximum(wid - (nw - r), 0) if r else wid * q
        nch = q + jnp.where(wid >= nw - r, 1, 0) if r else q
        base8 = (c0 // 8) * 8
        off = c0 - base8
        pltpu.sync_copy(dst2d_h.at[pl.ds(base8, qload)], idxall)
        plsc.subcore_barrier()
        _scatter_loop(p_h, acc, idxall, dbuf0, dbuf1, sem0, sem1,
                      c0, off, nch)
        plsc.subcore_barrier()
        _acc_writeout(acc, out_h, cid, sid, n, zrows)

    return sca(p128, dst2d, zeros)


def _sc_scatter128(ml, mr, dst2d, n):
    e, hd = ml.shape
    nc, ns = _sc_info()
    nw = nc * ns
    nch_total = e // 128
    q, r = divmod(nch_total, ns)
    qload = ((q + (1 if r else 0) + 8 + 7) // 8) * 8
    npad = ((n + nw * 8 - 1) // (nw * 8)) * (nw * 8)
    zrows = npad // ns
    zeros = jnp.zeros((zrows, hd), jnp.float32)
    mesh = plsc.VectorSubcoreMesh(core_axis_name="c", subcore_axis_name="s")

    @functools.partial(
        pl.kernel,
        out_type=jax.ShapeDtypeStruct((nc, n, hd), jnp.float32),
        mesh=mesh,
        scratch_types=[
            pltpu.VMEM((qload, 128), jnp.int32),
            pltpu.VMEM((128, hd), jnp.float32),
            pltpu.VMEM((128, hd), jnp.float32),
            pltpu.VMEM_SHARED((npad, hd), jnp.float32),
            pltpu.SemaphoreType.DMA,
            pltpu.SemaphoreType.DMA,
        ],
    )
    def sca(ml_h, mr_h, dst2d_h, z_h, out_h, idxall, dbuf0, dbuf1, acc,
            sem0, sem1):
        cid = lax.axis_index("c")
        sid = lax.axis_index("s")
        pltpu.sync_copy(z_h, acc.at[pl.ds(sid * zrows, zrows)])
        c0 = sid * q + jnp.maximum(sid - (ns - r), 0) if r else sid * q
        nch = q + jnp.where(sid >= ns - r, 1, 0) if r else q
        base8 = (c0 // 8) * 8
        off = c0 - base8
        pltpu.sync_copy(dst2d_h.at[pl.ds(base8, qload)], idxall)
        plsc.subcore_barrier()

        @pl.when(cid == 0)
        def _():
            _scatter_loop(ml_h, acc, idxall, dbuf0, dbuf1,
                          sem0, sem1, c0, off, nch)

        @pl.when(cid == 1)
        def _():
            _scatter_loop(mr_h, acc, idxall, dbuf0, dbuf1,
                          sem0, sem1, c0, off, nch)

        plsc.subcore_barrier()
        _acc_writeout(acc, out_h, cid, sid, n, zrows)

    return sca(ml, mr, dst2d, zeros)


def kernel(x, edge_attr, edge_index, Qw, Qb, Kw, Ew, Eb, Vw, Aw, VeRow,
           deg_coef, Ohw, Ohb, Oew, Oeb, bn1h_g, bn1h_b, bn1e_g, bn1e_b,
           W1, b1, W2, b2, bn2h_g, bn2h_b):
    n, d = x.shape
    e = edge_attr.shape[0]
    dh, h, _ = Aw.shape
    f32 = jnp.float32

    src = edge_index[0]
    dst = edge_index[1]

    w3 = jnp.concatenate([Qw, Kw, Vw], axis=1)
    qb2 = Qb.reshape(1, d)
    ewp = Ew.reshape(d, h, 2, dh).transpose(0, 2, 1, 3).reshape(d, 2 * d)
    ebp = Eb.reshape(h, 2, dh).transpose(1, 0, 2).reshape(1, 2 * d)
    eye_h = jnp.eye(h, dtype=f32)
    a2 = (Aw[:, :, 0][:, :, None] * eye_h[None, :, :]).transpose(1, 0, 2).reshape(d, h)
    a2p = jnp.concatenate([a2, jnp.zeros((d, 128 - h), f32)], axis=1)
    rb8 = jnp.kron(eye_h, jnp.ones((1, dh), f32))
    rb = jnp.concatenate([rb8, jnp.zeros((128 - h, d), f32)], axis=0)
    d8 = jnp.zeros((128, d), f32).at[h].set(1.0)
    vr2 = (VeRow[:, :, None, :] * eye_h[None, :, :, None]).transpose(1, 0, 2, 3).reshape(d, d)
    c0 = deg_coef[0, :, 0].reshape(1, d)
    c1 = deg_coef[0, :, 1].reshape(1, d)
    oebr = Oeb.reshape(1, d)
    ohbr = Ohb.reshape(1, d)

    bn = 1000
    gn = n // bn
    qh, kh, vh = pl.pallas_call(
        _qkv_body,
        grid=(gn,),
        in_specs=[pl.BlockSpec((bn, d), lambda i: (i, 0)),
                  pl.BlockSpec((d, 3 * d), lambda i: (0, 0)),
                  pl.BlockSpec((1, d), lambda i: (0, 0))],
        out_specs=[pl.BlockSpec((bn, d), lambda i: (i, 0))] * 3,
        out_shape=[jax.ShapeDtypeStruct((n, d), f32)] * 3,
    )(x, w3, qb2)

    pad8 = jnp.zeros((8, 64), jnp.int32)
    src2d = jnp.concatenate([src.reshape(e // 64, 64), pad8], axis=0)
    dst2dg = jnp.concatenate([dst.reshape(e // 64, 64), pad8], axis=0)
    kq, vs = _sc_gather(qh, kh, vh, src2d, dst2dg)

    be = 1280
    ge = e // be
    hd = d // 2
    e1, p128, msgl, msgr, rtl, rtr, bnpe = pl.pallas_call(
        _edge_body,
        grid=(ge,),
        in_specs=[pl.BlockSpec((be, d), lambda i: (i, 0)),
                  pl.BlockSpec((be, d), lambda i: (i, 0)),
                  pl.BlockSpec((be, d), lambda i: (i, 0)),
                  pl.BlockSpec((d, 2 * d), lambda i: (0, 0)),
                  pl.BlockSpec((1, 2 * d), lambda i: (0, 0)),
                  pl.BlockSpec((d, d), lambda i: (0, 0)),
                  pl.BlockSpec((1, d), lambda i: (0, 0)),
                  pl.BlockSpec((d, 128), lambda i: (0, 0)),
                  pl.BlockSpec((128, d), lambda i: (0, 0))],
        out_specs=[pl.BlockSpec((be, d), lambda i: (i, 0)),
                   pl.BlockSpec((be, 128), lambda i: (i, 0)),
                   pl.BlockSpec((be, hd), lambda i: (i, 0)),
                   pl.BlockSpec((be, hd), lambda i: (i, 0)),
                   pl.BlockSpec((be, hd), lambda i: (i, 0)),
                   pl.BlockSpec((be, hd), lambda i: (i, 0)),
                   pl.BlockSpec((1, 2, d), lambda i: (i, 0, 0))],
        out_shape=[jax.ShapeDtypeStruct((e, d), f32),
                   jax.ShapeDtypeStruct((e, 128), f32),
                   jax.ShapeDtypeStruct((e, hd), f32),
                   jax.ShapeDtypeStruct((e, hd), f32),
                   jax.ShapeDtypeStruct((e, hd), f32),
                   jax.ShapeDtypeStruct((e, hd), f32),
                   jax.ShapeDtypeStruct((ge, 2, d), f32)],
    )(edge_attr, kq, vs, ewp, ebp, Oew, oebr, a2p, rb)

    dst2d = jnp.concatenate(
        [dst.reshape(e // 128, 128),
         jnp.zeros((8, 128), jnp.int32)], axis=0)
    acc16 = _sc_scatter_part(p128, dst2d, n)
    accv2 = _sc_scatter128(msgl, msgr, dst2d, n)
    accr2 = _sc_scatter128(rtl, rtr, dst2d, n)
    accv = jnp.concatenate([accv2[0], accv2[1]], axis=1)
    accr = jnp.concatenate([accr2[0], accr2[1]], axis=1)

    ss_e = pl.pallas_call(
        functools.partial(_bnstats_body, float(e)),
        grid=(1,),
        in_specs=[pl.BlockSpec((ge, 2, d), lambda i: (0, 0, 0)),
                  pl.BlockSpec((1, d), lambda i: (0, 0)),
                  pl.BlockSpec((1, d), lambda i: (0, 0))],
        out_specs=pl.BlockSpec((2, d), lambda i: (0, 0)),
        out_shape=jax.ShapeDtypeStruct((2, d), f32),
    )(bnpe, bn1e_g.reshape(1, d), bn1e_b.reshape(1, d))
    e_out = pl.pallas_call(
        _bnapply_body,
        grid=(ge,),
        in_specs=[pl.BlockSpec((be, d), lambda i: (i, 0)),
                  pl.BlockSpec((2, d), lambda i: (0, 0))],
        out_specs=pl.BlockSpec((be, d), lambda i: (i, 0)),
        out_shape=jax.ShapeDtypeStruct((e, d), f32),
    )(e1, ss_e)

    h1, bnp1 = pl.pallas_call(
        _f1_body,
        grid=(gn,),
        in_specs=[pl.BlockSpec((bn, d), lambda i: (i, 0)),
                  pl.BlockSpec((bn, 128), lambda i: (i, 0)),
                  pl.BlockSpec((bn, 128), lambda i: (i, 0)),
                  pl.BlockSpec((bn, d), lambda i: (i, 0)),
                  pl.BlockSpec((bn, d), lambda i: (i, 0)),
                  pl.BlockSpec((d, d), lambda i: (0, 0)),
                  pl.BlockSpec((d, d), lambda i: (0, 0)),
                  pl.BlockSpec((1, d), lambda i: (0, 0)),
                  pl.BlockSpec((1, d), lambda i: (0, 0)),
                  pl.BlockSpec((1, d), lambda i: (0, 0)),
                  pl.BlockSpec((128, d), lambda i: (0, 0)),
                  pl.BlockSpec((128, d), lambda i: (0, 0))],
        out_specs=[pl.BlockSpec((bn, d), lambda i: (i, 0)),
                   pl.BlockSpec((1, 2, d), lambda i: (i, 0, 0))],
        out_shape=[jax.ShapeDtypeStruct((n, d), f32),
                   jax.ShapeDtypeStruct((gn, 2, d), f32)],
    )(x, acc16[0], acc16[1], accv, accr, vr2, Ohw, ohbr, c0, c1, rb, d8)

    f_mid, bnp2 = pl.pallas_call(
        functools.partial(_f2_body, float(n)),
        grid=(gn,),
        in_specs=[pl.BlockSpec((bn, d), lambda i: (i, 0)),
                  pl.BlockSpec((gn, 2, d), lambda i: (0, 0, 0)),
                  pl.BlockSpec((1, d), lambda i: (0, 0)),
                  pl.BlockSpec((1, d), lambda i: (0, 0)),
                  pl.BlockSpec((d, 2 * d), lambda i: (0, 0)),
                  pl.BlockSpec((1, 2 * d), lambda i: (0, 0)),
                  pl.BlockSpec((2 * d, d), lambda i: (0, 0)),
                  pl.BlockSpec((1, d), lambda i: (0, 0))],
        out_specs=[pl.BlockSpec((bn, d), lambda i: (i, 0)),
                   pl.BlockSpec((1, 2, d), lambda i: (i, 0, 0))],
        out_shape=[jax.ShapeDtypeStruct((n, d), f32),
                   jax.ShapeDtypeStruct((gn, 2, d), f32)],
    )(h1, bnp1, bn1h_g.reshape(1, d), bn1h_b.reshape(1, d), W1,
      b1.reshape(1, 2 * d), W2, b2.reshape(1, d))

    h_out = pl.pallas_call(
        functools.partial(_f3_body, float(n)),
        grid=(gn,),
        in_specs=[pl.BlockSpec((bn, d), lambda i: (i, 0)),
                  pl.BlockSpec((gn, 2, d), lambda i: (0, 0, 0)),
                  pl.BlockSpec((1, d), lambda i: (0, 0)),
                  pl.BlockSpec((1, d), lambda i: (0, 0))],
        out_specs=pl.BlockSpec((bn, d), lambda i: (i, 0)),
        out_shape=jax.ShapeDtypeStruct((n, d), f32),
    )(f_mid, bnp2, bn2h_g.reshape(1, d), bn2h_b.reshape(1, d))

    return h_out, e_out
